# baseline (speedup 1.0000x reference)
import functools

import jax
import jax.numpy as jnp
from jax import lax
from jax.experimental import pallas as pl
from jax.experimental.pallas import tpu as pltpu
from jax.experimental.pallas import tpu_sc as plsc

N_EMBD = 64
LANES = 16
NC = 2
NS = 16
NW = NC * NS

CH = 512
SUB = 128
NSUB = CH // SUB


def _body(x_hbm, table_hbm, out_hbm, idx_v, rows_v, gsem):
    wid = lax.axis_index("s") * NC + lax.axis_index("c")
    b_per_w = x_hbm.shape[0] // NW
    nch = b_per_w // CH
    base = wid * b_per_w

    def chunk_body(g, carry):
        cbase = base + g * CH
        pltpu.sync_copy(x_hbm.at[pl.ds(cbase, CH)], idx_v)
        copies = []
        for j in range(NSUB):
            copies.append(
                pltpu.async_copy(
                    table_hbm.at[idx_v.at[pl.ds(j * SUB, SUB)]],
                    rows_v.at[pl.ds(j * SUB, SUB)],
                    gsem,
                )
            )
        for c in copies:
            c.wait()

        ii = lax.iota(jnp.int32, 16)

        def row_body(r, rcarry):
            va = rows_v[r, pl.ds(0, LANES)]
            vb = rows_v[r, pl.ds(LANES, LANES)]
            vc = rows_v[r, pl.ds(2 * LANES, LANES)]
            vd = rows_v[r, pl.ds(3 * LANES, LANES)]
            s = va * va + vb * vb + vc * vc + vd * vd
            for step in (8, 4, 2, 1):
                s = s + s.at[ii ^ step].get(mode="promise_in_bounds")
            tot = s
            i = lax.bitcast_convert_type(tot, jnp.int32)
            i = jnp.full((LANES,), 0x5F3759DF, jnp.int32) - lax.shift_right_logical(i, 1)
            y = lax.bitcast_convert_type(i, jnp.float32)
            h = 0.5 * tot
            y = y * (1.5 - h * y * y)
            y = y * (1.5 - h * y * y)
            y = y * (1.5 - h * y * y)
            rows_v[r, pl.ds(0, LANES)] = va * y
            rows_v[r, pl.ds(LANES, LANES)] = vb * y
            rows_v[r, pl.ds(2 * LANES, LANES)] = vc * y
            rows_v[r, pl.ds(3 * LANES, LANES)] = vd * y
            return rcarry

        lax.fori_loop(0, CH, row_body, 0)
        pltpu.sync_copy(rows_v, out_hbm.at[pl.ds(cbase, CH)])
        return carry

    lax.fori_loop(0, nch, chunk_body, 0)


def kernel(x, table):
    B = x.shape[0] * x.shape[1]
    xf = jnp.reshape(x, (B,)).astype(jnp.int32)
    mesh = plsc.VectorSubcoreMesh(core_axis_name="c", subcore_axis_name="s")
    run = functools.partial(
        pl.kernel,
        out_type=jax.ShapeDtypeStruct((B, N_EMBD), jnp.float32),
        mesh=mesh,
        scratch_types=[
            pltpu.VMEM((CH,), jnp.int32),
            pltpu.VMEM((CH, N_EMBD), jnp.float32),
            pltpu.SemaphoreType.DMA,
        ],
        compiler_params=pltpu.CompilerParams(use_tc_tiling_on_sc=False),
    )(_body)
    out = run(xf, table)
    return jnp.reshape(out, (x.shape[0], x.shape[1], N_EMBD))


# speedup vs baseline: 1.5677x; 1.5677x over previous
import functools

import jax
import jax.numpy as jnp
from jax import lax
from jax.experimental import pallas as pl
from jax.experimental.pallas import tpu as pltpu
from jax.experimental.pallas import tpu_sc as plsc

N_EMBD = 64
LANES = 16
NC = 2
NS = 16
NW = NC * NS

CH = 512
SUB = 128
NSUB = CH // SUB
UNROLL = 4


def _fire_gather(table_hbm, idx_all, rows, sem, g):
    for j in range(NSUB):
        pltpu.async_copy(
            table_hbm.at[idx_all.at[pl.ds(g * CH + j * SUB, SUB)]],
            rows.at[pl.ds(j * SUB, SUB)],
            sem,
        )


def _wait_gather(table_hbm, idx_all, rows, sem):
    for j in range(NSUB):
        pltpu.make_async_copy(
            table_hbm.at[idx_all.at[pl.ds(j * SUB, SUB)]],
            rows.at[pl.ds(j * SUB, SUB)],
            sem,
        ).wait()


def _wait_out(rows, out_hbm, sem):
    pltpu.make_async_copy(rows, out_hbm.at[pl.ds(0, CH)], sem).wait()


def _compute(rows):
    ii = lax.iota(jnp.int32, LANES)

    def quad(r, rcarry):
        rb = r * UNROLL
        for k in range(UNROLL):
            row = rb + k
            va = rows[row, pl.ds(0, LANES)]
            vb = rows[row, pl.ds(LANES, LANES)]
            vc = rows[row, pl.ds(2 * LANES, LANES)]
            vd = rows[row, pl.ds(3 * LANES, LANES)]
            s = va * va + vb * vb + vc * vc + vd * vd
            for step in (8, 4, 2, 1):
                s = s + s.at[ii ^ step].get(mode="promise_in_bounds")
            i = lax.bitcast_convert_type(s, jnp.int32)
            i = jnp.full((LANES,), 0x5F3759DF, jnp.int32) - lax.shift_right_logical(i, 1)
            y = lax.bitcast_convert_type(i, jnp.float32)
            h = 0.5 * s
            y = y * (1.5 - h * y * y)
            y = y * (1.5 - h * y * y)
            y = y * (1.5 - h * y * y)
            rows[row, pl.ds(0, LANES)] = va * y
            rows[row, pl.ds(LANES, LANES)] = vb * y
            rows[row, pl.ds(2 * LANES, LANES)] = vc * y
            rows[row, pl.ds(3 * LANES, LANES)] = vd * y
        return rcarry

    lax.fori_loop(0, CH // UNROLL, quad, 0)


def _body(x_hbm, table_hbm, out_hbm, idx_all, rows0, rows1,
          gsem0, gsem1, osem0, osem1):
    wid = lax.axis_index("s") * NC + lax.axis_index("c")
    b_per_w = x_hbm.shape[0] // NW
    nch = b_per_w // CH
    base = wid * b_per_w

    rows = (rows0, rows1)
    gsem = (gsem0, gsem1)
    osem = (osem0, osem1)

    pltpu.sync_copy(x_hbm.at[pl.ds(base, b_per_w)], idx_all)

    _fire_gather(table_hbm, idx_all, rows0, gsem0, 0)
    _fire_gather(table_hbm, idx_all, rows1, gsem1, 1)
    _wait_gather(table_hbm, idx_all, rows0, gsem0)
    _compute(rows0)
    pltpu.async_copy(rows0, out_hbm.at[pl.ds(base, CH)], osem0)

    def pair(i, carry):
        for off in range(2):
            g = 1 + 2 * i + off
            b = (1 + off) % 2
            nb = 1 - b
            _wait_out(rows[nb], out_hbm, osem[nb])
            _fire_gather(table_hbm, idx_all, rows[nb], gsem[nb], g + 1)
            _wait_gather(table_hbm, idx_all, rows[b], gsem[b])
            _compute(rows[b])
            pltpu.async_copy(rows[b], out_hbm.at[pl.ds(base + g * CH, CH)], osem[b])
        return carry

    lax.fori_loop(0, (nch - 2) // 2, pair, 0)

    gl = nch - 1
    bl = gl % 2
    _wait_gather(table_hbm, idx_all, rows[bl], gsem[bl])
    _compute(rows[bl])
    pltpu.async_copy(rows[bl], out_hbm.at[pl.ds(base + gl * CH, CH)], osem[bl])
    _wait_out(rows[0], out_hbm, osem[0])
    _wait_out(rows[1], out_hbm, osem[1])


def kernel(x, table):
    B = x.shape[0] * x.shape[1]
    b_per_w = B // NW
    nch = b_per_w // CH
    assert B % NW == 0 and b_per_w % CH == 0 and nch % 2 == 0 and nch >= 4
    xf = jnp.reshape(x, (B,)).astype(jnp.int32)
    mesh = plsc.VectorSubcoreMesh(core_axis_name="c", subcore_axis_name="s")
    run = functools.partial(
        pl.kernel,
        out_type=jax.ShapeDtypeStruct((B, N_EMBD), jnp.float32),
        mesh=mesh,
        scratch_types=[
            pltpu.VMEM((b_per_w,), jnp.int32),
            pltpu.VMEM((CH, N_EMBD), jnp.float32),
            pltpu.VMEM((CH, N_EMBD), jnp.float32),
            pltpu.SemaphoreType.DMA,
            pltpu.SemaphoreType.DMA,
            pltpu.SemaphoreType.DMA,
            pltpu.SemaphoreType.DMA,
        ],
        compiler_params=pltpu.CompilerParams(use_tc_tiling_on_sc=False),
    )(_body)
    out = run(xf, table)
    return jnp.reshape(out, (x.shape[0], x.shape[1], N_EMBD))
